# baseline (speedup 1.0000x reference)
import functools

import jax
import jax.numpy as jnp
from jax import lax
from jax.experimental import pallas as pl
from jax.experimental.pallas import tpu as pltpu
from jax.experimental.pallas import tpu_sc as plsc

_DIM = 64
_IDX_LANES = 128
_HALF = 256


@functools.cache
def _build_gather(batch, fields):
    info = plsc.get_sparse_core_info()
    nc, ns = info.num_cores, info.num_subcores
    nw = nc * ns
    bpw = batch // nw
    halves = bpw // _HALF
    steps = fields * halves
    tiles_b = batch // 128
    tiles_half = _HALF // 128

    mesh = plsc.VectorSubcoreMesh(core_axis_name="c", subcore_axis_name="s")

    @functools.partial(
        pl.kernel,
        mesh=mesh,
        compiler_params=pltpu.CompilerParams(
            use_tc_tiling_on_sc=False, needs_layout_passes=False
        ),
        out_type=jax.ShapeDtypeStruct(
            (fields, _DIM // 8, tiles_b, 8, 128), jnp.float32
        ),
        scratch_types=[
            pltpu.VMEM((fields, bpw), jnp.int32),
            pltpu.VMEM((2, _HALF, _DIM), jnp.float32),
            pltpu.VMEM((2, _DIM // 8, tiles_half, 8, 128), jnp.float32),
            pltpu.SemaphoreType.DMA((2,)),
            pltpu.SemaphoreType.DMA((2,)),
        ],
    )
    def gather_kernel(xt_hbm, table_hbm, out_hbm, idx_v, rv, tv, gsem, ssem):
        wid = lax.axis_index("s") * nc + lax.axis_index("c")
        b0 = wid * bpw
        bg0 = wid * (bpw // 128)
        pltpu.sync_copy(xt_hbm.at[:, pl.ds(b0, bpw)], idx_v)

        iota = lax.iota(jnp.int32, 16)

        def start_gather(f, half, p):
            for j in range(_HALF // _IDX_LANES):
                pltpu.async_copy(
                    table_hbm.at[
                        idx_v.at[f, pl.ds(half * _HALF + j * _IDX_LANES,
                                          _IDX_LANES)]
                    ],
                    rv.at[p, pl.ds(j * _IDX_LANES, _IDX_LANES)],
                    gsem.at[p],
                )

        def wait_gather(p):
            pltpu.make_async_copy(
                table_hbm.at[pl.ds(0, _HALF)], rv.at[p], gsem.at[p]
            ).wait()

        def store_dst(f, half):
            return out_hbm.at[f, :, pl.ds(bg0 + half * tiles_half, tiles_half)]

        def wait_store(p):
            pltpu.make_async_copy(tv.at[p], store_dst(0, 0), ssem.at[p]).wait()

        start_gather(0, 0, 0)

        @pl.loop(0, steps, step=2)
        def _steps(lv):
            f = lv // halves
            for p in range(2):
                s = lv + p

                @pl.when(s < steps - 1)
                def _():
                    start_gather((s + 1) // halves, (s + 1) % halves, 1 - p)

                wait_gather(p)

                @pl.when(s >= 2)
                def _():
                    wait_store(p)

                @pl.loop(0, _DIM, unroll=8)
                def _cols(c):
                    cg = c // 8
                    ci = c % 8
                    csplat = jnp.full((16,), c, jnp.int32)
                    for l in range(tiles_half):
                        for k in range(128 // 16):
                            vals = plsc.load_gather(
                                rv.at[p],
                                [iota + (l * 128 + k * 16), csplat],
                            )
                            tv[p, cg, l, ci, pl.ds(k * 16, 16)] = vals

                pltpu.async_copy(tv.at[p], store_dst(f, p), ssem.at[p])

        for p in range(2):
            wait_store(p)

    return gather_kernel


def kernel(x, weight):
    b, f = x.shape
    xt = jnp.swapaxes(x, 0, 1).astype(jnp.int32)
    out5 = _build_gather(b, f)(xt, weight)
    return jnp.transpose(out5, (2, 4, 0, 1, 3)).reshape(b, f, _DIM)


# speedup vs baseline: 1.4221x; 1.4221x over previous
import functools

import jax
import jax.numpy as jnp
from jax import lax
from jax.experimental import pallas as pl
from jax.experimental.pallas import tpu as pltpu
from jax.experimental.pallas import tpu_sc as plsc

_DIM = 64
_IDX_LANES = 128
_CHUNK = 512
_NBUF = 3
_BBLK = 128


@functools.cache
def _build_gather(n_total):
    info = plsc.get_sparse_core_info()
    nc, ns = info.num_cores, info.num_subcores
    nw = nc * ns
    rows_per_w = n_total // nw
    n_chunks = rows_per_w // _CHUNK
    k = _CHUNK // _IDX_LANES
    idx_rows_per_w = rows_per_w // _IDX_LANES

    mesh = plsc.VectorSubcoreMesh(core_axis_name="c", subcore_axis_name="s")

    @functools.partial(
        pl.kernel,
        mesh=mesh,
        compiler_params=pltpu.CompilerParams(use_tc_tiling_on_sc=False),
        out_type=jax.ShapeDtypeStruct((n_total, _DIM), jnp.float32),
        scratch_types=[
            pltpu.VMEM((idx_rows_per_w, _IDX_LANES), jnp.int32),
            pltpu.VMEM((_NBUF, _CHUNK, _DIM), jnp.float32),
            pltpu.SemaphoreType.DMA((_NBUF,)),
            pltpu.SemaphoreType.DMA((_NBUF,)),
        ],
    )
    def gather_kernel(idx_hbm, table_hbm, out_hbm, idx_all, rows_v, gsem, ssem):
        wid = lax.axis_index("s") * nc + lax.axis_index("c")
        idx_row0 = wid * idx_rows_per_w
        out_row0 = wid * rows_per_w
        pltpu.sync_copy(idx_hbm.at[pl.ds(idx_row0, idx_rows_per_w)], idx_all)

        def start_gathers(c, b):
            return [
                pltpu.async_copy(
                    table_hbm.at[idx_all.at[c * k + j]],
                    rows_v.at[b, pl.ds(j * _IDX_LANES, _IDX_LANES)],
                    gsem.at[b],
                )
                for j in range(k)
            ]

        def start_store(c, b):
            return pltpu.async_copy(
                rows_v.at[b],
                out_hbm.at[pl.ds(out_row0 + c * _CHUNK, _CHUNK)],
                ssem.at[b],
            )

        stores = {}
        pend = {}
        for c in range(n_chunks):
            b = c % _NBUF
            if c >= _NBUF:
                stores.pop(b).wait()
            pend[b] = start_gathers(c, b)
            if c >= 1:
                bp = (c - 1) % _NBUF
                for cp in pend.pop(bp):
                    cp.wait()
                stores[bp] = start_store(c - 1, bp)
        blast = (n_chunks - 1) % _NBUF
        for cp in pend.pop(blast):
            cp.wait()
        stores[blast] = start_store(n_chunks - 1, blast)
        for b in list(stores):
            stores.pop(b).wait()

    return gather_kernel


@functools.cache
def _build_transpose(batch, fields):
    n_total = batch * fields
    m_total = n_total * _DIM // 128
    m_per_blk = _BBLK * fields * _DIM // 128

    del n_total, m_total, m_per_blk
    fd = fields * _DIM

    def body(rows_ref, out_ref):
        t = jnp.transpose(rows_ref[...])
        out_ref[...] = t.reshape(fields, _DIM // 8, 1, 8, _BBLK)

    return pl.pallas_call(
        body,
        grid=(batch // _BBLK,),
        in_specs=[
            pl.BlockSpec((_BBLK, fd), lambda i: (i, 0)),
        ],
        out_specs=pl.BlockSpec(
            (fields, _DIM // 8, 1, 8, _BBLK), lambda i: (0, 0, i, 0, 0)
        ),
        out_shape=jax.ShapeDtypeStruct(
            (fields, _DIM // 8, batch // _BBLK, 8, _BBLK), jnp.float32
        ),
    )


def kernel(x, weight):
    b, f = x.shape
    n_total = b * f
    lanes_f = f * _DIM // 128
    xp = (
        x.astype(jnp.int32)
        .reshape(b // 8, 8, lanes_f, 2)
        .transpose(0, 2, 1, 3)
        .reshape(-1)
    )
    idx2d = xp.reshape(n_total // _IDX_LANES, _IDX_LANES)
    rows = _build_gather(n_total)(idx2d, weight)
    rows2d = (
        rows.reshape(b // 8, lanes_f, 8, 128)
        .transpose(0, 2, 1, 3)
        .reshape(b, f * _DIM)
    )
    out5 = _build_transpose(b, f)(rows2d)
    return jnp.transpose(out5, (2, 4, 0, 1, 3)).reshape(b, f, _DIM)


# speedup vs baseline: 1.5795x; 1.1107x over previous
import functools

import jax
import jax.numpy as jnp
from jax import lax
from jax.experimental import pallas as pl
from jax.experimental.pallas import tpu as pltpu
from jax.experimental.pallas import tpu_sc as plsc

_DIM = 64
_IDX_LANES = 128
_CHUNK = 128
_NBUF = 4
_BBLK = 128


@functools.cache
def _build_gather(n_total):
    info = plsc.get_sparse_core_info()
    nc, ns = info.num_cores, info.num_subcores
    nw = nc * ns
    rows_per_w = n_total // nw
    n_chunks = rows_per_w // _CHUNK
    idx_rows_per_w = rows_per_w // _IDX_LANES

    mesh = plsc.VectorSubcoreMesh(core_axis_name="c", subcore_axis_name="s")

    @functools.partial(
        pl.kernel,
        mesh=mesh,
        compiler_params=pltpu.CompilerParams(use_tc_tiling_on_sc=True),
        out_type=jax.ShapeDtypeStruct((n_total, _DIM), jnp.float32),
        scratch_types=[
            pltpu.VMEM((idx_rows_per_w, _IDX_LANES), jnp.int32),
            pltpu.VMEM((_NBUF, _CHUNK, _DIM), jnp.float32),
            pltpu.SemaphoreType.DMA((_NBUF,)),
            pltpu.SemaphoreType.DMA((_NBUF,)),
        ],
    )
    def gather_kernel(idx_hbm, table_hbm, out_hbm, idx_all, rows_v, gsem, ssem):
        wid = lax.axis_index("s") * nc + lax.axis_index("c")
        idx_row0 = wid * idx_rows_per_w
        out_row0 = wid * rows_per_w
        pltpu.sync_copy(idx_hbm.at[pl.ds(idx_row0, idx_rows_per_w)], idx_all)

        def start_gathers(c, b):
            @pl.loop(0, _CHUNK // 16)
            def _grp(g):
                vec = idx_all[c, pl.ds(g * 16, 16)]
                for i in range(16):
                    pltpu.async_copy(
                        table_hbm.at[pl.ds(vec[i], 1)],
                        rows_v.at[b, pl.ds(g * 16 + i, 1)],
                        gsem.at[b],
                    )

        def wait_gathers(b):
            pltpu.make_async_copy(
                table_hbm.at[pl.ds(0, _CHUNK)], rows_v.at[b], gsem.at[b]
            ).wait()

        def start_store(c, b):
            return pltpu.async_copy(
                rows_v.at[b],
                out_hbm.at[pl.ds(out_row0 + c * _CHUNK, _CHUNK)],
                ssem.at[b],
            )

        def wait_store(b):
            pltpu.make_async_copy(
                rows_v.at[b], out_hbm.at[pl.ds(out_row0, _CHUNK)], ssem.at[b]
            ).wait()

        start_gathers(0, 0)

        @pl.loop(0, n_chunks, step=_NBUF)
        def _chunks(lv):
            for p in range(_NBUF):
                c = lv + p
                c_fin = c - 2
                bf = (p - 2) % _NBUF

                @pl.when(c_fin >= 0)
                def _():
                    wait_gathers(bf)
                    start_store(c_fin, bf)

                c_new = c + 1
                bn = (p + 1) % _NBUF

                @pl.when(c_new < n_chunks)
                def _():
                    @pl.when(c_new >= _NBUF)
                    def _():
                        wait_store(bn)

                    start_gathers(c_new, bn)

        for c_fin in (n_chunks - 2, n_chunks - 1):
            bf = c_fin % _NBUF
            wait_gathers(bf)
            start_store(c_fin, bf)
        for b in range(_NBUF):
            wait_store(b)

    return gather_kernel


@functools.cache
def _build_transpose(batch, fields):
    rows_per_blk = _BBLK * fields

    def body(rows_ref, out_ref):
        a = rows_ref[...].reshape(_BBLK, fields, _DIM)
        t = jnp.transpose(a, (1, 2, 0))
        out_ref[...] = t.reshape(fields, _DIM // 8, 1, 8, _BBLK)

    return pl.pallas_call(
        body,
        grid=(batch // _BBLK,),
        in_specs=[
            pl.BlockSpec((rows_per_blk, _DIM), lambda i: (i, 0)),
        ],
        out_specs=pl.BlockSpec(
            (fields, _DIM // 8, 1, 8, _BBLK), lambda i: (0, 0, i, 0, 0)
        ),
        out_shape=jax.ShapeDtypeStruct(
            (fields, _DIM // 8, batch // _BBLK, 8, _BBLK), jnp.float32
        ),
    )


def kernel(x, weight):
    b, f = x.shape
    n_total = b * f
    idx2d = x.reshape(n_total // _IDX_LANES, _IDX_LANES).astype(jnp.int32)
    rows = _build_gather(n_total)(idx2d, weight)
    out5 = _build_transpose(b, f)(rows)
    return jnp.transpose(out5, (2, 4, 0, 1, 3)).reshape(b, f, _DIM)
